# baseline (speedup 1.0000x reference)
import jax
import jax.numpy as jnp
from jax.experimental import pallas as pl

_B = 1024
_M = 8
_K = 32
_D = 128
_OUT = 128
_TR = 512


def _body(sv_ref, nb_ref, mk_ref, w_ref, b_ref, out_ref):
    g = pl.program_id(1)
    x = nb_ref[:, 0]
    scale = 1.0 / _K
    dn = (((1,), (1,)), ((), ()))

    @pl.when(g == 0)
    def _():
        e = jnp.sum(x * mk_ref[...][:, :_K, None], axis=1) * scale
        out_ref[...] = (
            jax.lax.dot_general(e, w_ref[:, _D:2 * _D], dn,
                                preferred_element_type=jnp.float32)
            + jax.lax.dot_general(sv_ref[...], w_ref[:, :_D], dn,
                                  preferred_element_type=jnp.float32)
            + b_ref[...])

    @pl.when(g == 1)
    def _():
        e = jnp.sum(x * mk_ref[...][:, _K:, None], axis=1) * scale
        out_ref[...] += jax.lax.dot_general(
            e, w_ref[:, 2 * _D:], dn, preferred_element_type=jnp.float32)


def kernel(self_vectors, neighbor_vectors, masks, W, b):
    R = _B * _M
    nb = neighbor_vectors.reshape(R, 2, _K, _D)
    mk = masks.reshape(R, 2 * _K)
    sv = self_vectors.reshape(R, _D)
    b2 = b.reshape(1, _OUT)

    grid = (R // _TR, 2)
    out = pl.pallas_call(
        _body,
        grid=grid,
        in_specs=[
            pl.BlockSpec((_TR, _D), lambda i, g: (i, 0)),
            pl.BlockSpec((_TR, 1, _K, _D), lambda i, g: (i, g, 0, 0)),
            pl.BlockSpec((_TR, 2 * _K), lambda i, g: (i, 0)),
            pl.BlockSpec((_OUT, 3 * _D), lambda i, g: (0, 0)),
            pl.BlockSpec((1, _OUT), lambda i, g: (0, 0)),
        ],
        out_specs=pl.BlockSpec((_TR, _OUT), lambda i, g: (i, 0)),
        out_shape=jax.ShapeDtypeStruct((R, _OUT), jnp.float32),
    )(sv, nb, mk, W, b2)
    return out.reshape(_B, _M, _OUT)


# speedup vs baseline: 1.1851x; 1.1851x over previous
import jax
import jax.numpy as jnp
from jax.experimental import pallas as pl

_B = 1024
_M = 8
_K = 32
_D = 128
_OUT = 128
_TR = 512


def _body(sv_ref, nb0_ref, nb1_ref, mk_ref, w_ref, b_ref, out_ref):
    x0 = nb0_ref[:, 0]
    x1 = nb1_ref[:, 0]
    m = mk_ref[...]
    w = w_ref[...]
    sv = sv_ref[...]

    scale = 1.0 / _K
    bdn = (((2,), (1,)), ((0,), (0,)))
    e0 = jax.lax.dot_general(m[:, None, :_K], x0, bdn,
                             preferred_element_type=jnp.float32)[:, 0] * scale
    e1 = jax.lax.dot_general(m[:, None, _K:], x1, bdn,
                             preferred_element_type=jnp.float32)[:, 0] * scale

    dn = (((1,), (1,)), ((), ()))
    acc = jax.lax.dot_general(sv, w[:, :_D], dn,
                              preferred_element_type=jnp.float32)
    acc += jax.lax.dot_general(e0, w[:, _D:2 * _D], dn,
                               preferred_element_type=jnp.float32)
    acc += jax.lax.dot_general(e1, w[:, 2 * _D:], dn,
                               preferred_element_type=jnp.float32)
    out_ref[...] = acc + b_ref[...]


def kernel(self_vectors, neighbor_vectors, masks, W, b):
    R = _B * _M
    nb = neighbor_vectors.reshape(R, 2, _K, _D)
    mk = masks.reshape(R, 2 * _K)
    sv = self_vectors.reshape(R, _D)
    b2 = b.reshape(1, _OUT)

    grid = (R // _TR,)
    out = pl.pallas_call(
        _body,
        grid=grid,
        in_specs=[
            pl.BlockSpec((_TR, _D), lambda i: (i, 0)),
            pl.BlockSpec((_TR, 1, _K, _D), lambda i: (i, 0, 0, 0)),
            pl.BlockSpec((_TR, 1, _K, _D), lambda i: (i, 1, 0, 0)),
            pl.BlockSpec((_TR, 2 * _K), lambda i: (i, 0)),
            pl.BlockSpec((_OUT, 3 * _D), lambda i: (0, 0)),
            pl.BlockSpec((1, _OUT), lambda i: (0, 0)),
        ],
        out_specs=pl.BlockSpec((_TR, _OUT), lambda i: (i, 0)),
        out_shape=jax.ShapeDtypeStruct((R, _OUT), jnp.float32),
    )(sv, nb, nb, mk, W, b2)
    return out.reshape(_B, _M, _OUT)
